# Initial kernel scaffold; baseline (speedup 1.0000x reference)
import jax
import jax.numpy as jnp
from jax import lax
from jax.experimental import pallas as pl
from jax.experimental.pallas import tpu as pltpu
from jax.experimental.pallas import tpu_sc as plsc

N_SRC = 100000
D = 128
NSEC = 100
NSEC_PAD = 128
B = 16384
SLICE_RANGE = 1000

NC = 2
NS = 16
NW = NC * NS

CHUNK = 400
SUB = 100
NSUB = CHUNK // SUB
NCHUNKS = N_SRC // CHUNK
NCH_PER_W = -(-NCHUNKS // NW)

CNT_W = 16

TBLK = 1024
GRID = B // TBLK


def _seg_sum_body(src_emb, sectors, zeros_acc, zeros_cnt, ones_h,
                  out_sums, out_cnts,
                  rows_v, idx_v, ones_v, acc_sh, cnt_sh):
    c = lax.axis_index("c")
    s = lax.axis_index("s")
    wid = c * NS + s

    @pl.when(s == 0)
    def _():
        pltpu.sync_copy(zeros_acc, acc_sh)
        pltpu.sync_copy(zeros_cnt, cnt_sh)

    pltpu.sync_copy(ones_h, ones_v)
    plsc.subcore_barrier()

    for k in range(NCH_PER_W):
        ci = k * NW + wid

        @pl.when(ci < NCHUNKS)
        def _(ci=ci):
            pltpu.sync_copy(src_emb.at[pl.ds(ci * CHUNK, CHUNK)], rows_v)
            pltpu.sync_copy(sectors.at[ci], idx_v)
            for j in range(NSUB):
                pltpu.sync_copy(rows_v.at[pl.ds(j * SUB, SUB)],
                                acc_sh.at[idx_v.at[j]], add=True)
                pltpu.sync_copy(ones_v, cnt_sh.at[idx_v.at[j]], add=True)

    plsc.subcore_barrier()

    @pl.when(s == 0)
    def _():
        pltpu.sync_copy(acc_sh, out_sums.at[c])
        pltpu.sync_copy(cnt_sh, out_cnts.at[c])


_seg_call = pl.kernel(
    _seg_sum_body,
    out_type=[
        jax.ShapeDtypeStruct((NC, NSEC_PAD, D), jnp.float32),
        jax.ShapeDtypeStruct((NC, NSEC_PAD, CNT_W), jnp.float32),
    ],
    mesh=plsc.VectorSubcoreMesh(core_axis_name="c", subcore_axis_name="s"),
    scratch_types=[
        pltpu.VMEM((CHUNK, D), jnp.float32),
        pltpu.VMEM((NSUB, SUB), jnp.int32),
        pltpu.VMEM((SUB, CNT_W), jnp.float32),
        pltpu.VMEM_SHARED((NSEC_PAD, D), jnp.float32),
        pltpu.VMEM_SHARED((NSEC_PAD, CNT_W), jnp.float32),
    ],
)


def _dist_body(sums_ref, cnts_ref, temb_ref, tidx_ref, out_ref, centers_scr):
    i = pl.program_id(0)

    @pl.when(i == 0)
    def _():
        ssum = sums_ref[0] + sums_ref[1]
        cnt = cnts_ref[0] + cnts_ref[1]
        centers = ssum / jnp.maximum(cnt, 1.0)[:, None]
        centers_scr[...] = centers.astype(jnp.bfloat16)
        out_ref[0, 0] = 0.0

    sec = tidx_ref[0, 0, :] // SLICE_RANGE
    onehot = (sec[:, None] == lax.broadcasted_iota(
        jnp.int32, (TBLK, NSEC_PAD), 1)).astype(jnp.bfloat16)
    cc = jnp.dot(onehot, centers_scr[...],
                 preferred_element_type=jnp.float32)
    diff = temb_ref[...] - cc + 1e-6
    dist = jnp.sqrt(jnp.sum(diff * diff, axis=-1))
    out_ref[0, 0] += jnp.sum(dist) * (1.0 / B)


def _dist_call(sums, cnts, temb, tidx_r):
    return pl.pallas_call(
        _dist_body,
        grid=(GRID,),
        in_specs=[
            pl.BlockSpec((NC, NSEC_PAD, D), lambda i: (0, 0, 0)),
            pl.BlockSpec((NC, NSEC_PAD), lambda i: (0, 0)),
            pl.BlockSpec((TBLK, D), lambda i: (i, 0)),
            pl.BlockSpec((1, 1, TBLK), lambda i: (i, 0, 0)),
        ],
        out_specs=pl.BlockSpec((1, 1), lambda i: (0, 0)),
        out_shape=jax.ShapeDtypeStruct((1, 1), jnp.float32),
        scratch_shapes=[pltpu.VMEM((NSEC_PAD, D), jnp.bfloat16)],
    )(sums, cnts, temb, tidx_r)


def kernel(target_embeddings, target_slice_idx, source_embeddings,
           source_slice_idx, source_sectors):
    del source_slice_idx
    sectors_r = source_sectors.astype(jnp.int32).reshape(NCHUNKS, NSUB, SUB)
    zeros_acc = jnp.zeros((NSEC_PAD, D), jnp.float32)
    zeros_cnt = jnp.zeros((NSEC_PAD, CNT_W), jnp.float32)
    ones_h = jnp.ones((SUB, CNT_W), jnp.float32)

    sums, cnts3 = _seg_call(source_embeddings, sectors_r,
                            zeros_acc, zeros_cnt, ones_h)
    cnts = cnts3[..., 0]

    tidx_r = target_slice_idx.astype(jnp.int32).reshape(GRID, 1, TBLK)
    out = _dist_call(sums, cnts, target_embeddings, tidx_r)
    return out[0, 0]


# speedup vs baseline: 6.6896x; 6.6896x over previous
import jax
import jax.numpy as jnp
from jax import lax
from jax.experimental import pallas as pl
from jax.experimental.pallas import tpu as pltpu
from jax.experimental.pallas import tpu_sc as plsc

N_SRC = 100000
D = 128
NSEC = 100
NSEC_PAD = 128
B = 16384
SLICE_RANGE = 1000

NC = 2
NS = 16
NW = NC * NS

CHUNK = 400
SUB = 100
NSUB = CHUNK // SUB
NCHUNKS = N_SRC // CHUNK
NCH_PER_W = -(-NCHUNKS // NW)

HBLK = 1024
HGRID = -(-N_SRC // HBLK)
N_SRC_PAD = HGRID * HBLK

TBLK = 1024
GRID = B // TBLK


def _seg_sum_body(src_emb, sectors, zeros_acc, out_sums,
                  rows_v, idx_v, acc_sh):
    c = lax.axis_index("c")
    s = lax.axis_index("s")
    wid = c * NS + s

    @pl.when(s == 0)
    def _():
        pltpu.sync_copy(zeros_acc, acc_sh)

    plsc.subcore_barrier()

    for k in range(NCH_PER_W):
        ci = k * NW + wid

        @pl.when(ci < NCHUNKS)
        def _(ci=ci):
            pltpu.sync_copy(src_emb.at[pl.ds(ci * CHUNK, CHUNK)], rows_v)
            pltpu.sync_copy(sectors.at[ci], idx_v)
            for j in range(NSUB):
                pltpu.sync_copy(rows_v.at[pl.ds(j * SUB, SUB)],
                                acc_sh.at[idx_v.at[j]], add=True)

    plsc.subcore_barrier()

    @pl.when(s == 0)
    def _():
        pltpu.sync_copy(acc_sh, out_sums.at[c])


_seg_call_cache = []


def _seg_call(*args):
    if not _seg_call_cache:
        _seg_call_cache.append(pl.kernel(
            _seg_sum_body,
            out_type=jax.ShapeDtypeStruct((NC, NSEC_PAD, D), jnp.float32),
            mesh=plsc.VectorSubcoreMesh(core_axis_name="c",
                                        subcore_axis_name="s",
                                        num_cores=NC, num_subcores=NS),
            scratch_types=[
                pltpu.VMEM((CHUNK, D), jnp.float32),
                pltpu.VMEM((NSUB, SUB), jnp.int32),
                pltpu.VMEM_SHARED((NSEC_PAD, D), jnp.float32),
            ],
        ))
    return _seg_call_cache[0](*args)


def _hist_body(sec_ref, out_ref):
    i = pl.program_id(0)

    @pl.when(i == 0)
    def _():
        out_ref[...] = jnp.zeros((1, NSEC_PAD), jnp.float32)

    sec = sec_ref[0, 0, :]
    onehot = (sec[:, None] == lax.broadcasted_iota(
        jnp.int32, (HBLK, NSEC_PAD), 1)).astype(jnp.float32)
    out_ref[...] += jnp.sum(onehot, axis=0).reshape(1, NSEC_PAD)


def _hist_call(sec_r):
    return pl.pallas_call(
        _hist_body,
        grid=(HGRID,),
        in_specs=[pl.BlockSpec((1, 1, HBLK), lambda i: (i, 0, 0))],
        out_specs=pl.BlockSpec((1, NSEC_PAD), lambda i: (0, 0)),
        out_shape=jax.ShapeDtypeStruct((1, NSEC_PAD), jnp.float32),
    )(sec_r)


def _dist_body(sums_ref, cnts_ref, temb_ref, tidx_ref, out_ref, centers_scr):
    i = pl.program_id(0)

    @pl.when(i == 0)
    def _():
        ssum = sums_ref[0] + sums_ref[1]
        cnt = cnts_ref[0]
        centers = ssum / jnp.maximum(cnt, 1.0)[:, None]
        centers_scr[...] = centers.astype(jnp.bfloat16)
        out_ref[...] = jnp.zeros((1, 1), jnp.float32)

    sec = tidx_ref[0, 0, :] // SLICE_RANGE
    onehot = (sec[:, None] == lax.broadcasted_iota(
        jnp.int32, (TBLK, NSEC_PAD), 1)).astype(jnp.bfloat16)
    cc = jnp.dot(onehot, centers_scr[...],
                 preferred_element_type=jnp.float32)
    diff = temb_ref[...] - cc + 1e-6
    dist = jnp.sqrt(jnp.sum(diff * diff, axis=-1))
    out_ref[...] += (jnp.sum(dist) * (1.0 / B)).reshape(1, 1)


def _dist_call(sums, cnts, temb, tidx_r):
    return pl.pallas_call(
        _dist_body,
        grid=(GRID,),
        in_specs=[
            pl.BlockSpec((NC, NSEC_PAD, D), lambda i: (0, 0, 0)),
            pl.BlockSpec((1, NSEC_PAD), lambda i: (0, 0)),
            pl.BlockSpec((TBLK, D), lambda i: (i, 0)),
            pl.BlockSpec((1, 1, TBLK), lambda i: (i, 0, 0)),
        ],
        out_specs=pl.BlockSpec((1, 1), lambda i: (0, 0)),
        out_shape=jax.ShapeDtypeStruct((1, 1), jnp.float32),
        scratch_shapes=[pltpu.VMEM((NSEC_PAD, D), jnp.bfloat16)],
    )(sums, cnts, temb, tidx_r)


def kernel(target_embeddings, target_slice_idx, source_embeddings,
           source_slice_idx, source_sectors):
    del source_slice_idx
    sec32 = source_sectors.astype(jnp.int32)
    sectors_r = sec32.reshape(NCHUNKS, NSUB, SUB)
    zeros_acc = jnp.zeros((NSEC_PAD, D), jnp.float32)

    sums = _seg_call(source_embeddings, sectors_r, zeros_acc)

    sec_pad = jnp.concatenate(
        [sec32, jnp.full((N_SRC_PAD - N_SRC,), NSEC_PAD - 1, jnp.int32)])
    cnts = _hist_call(sec_pad.reshape(HGRID, 1, HBLK))

    tidx_r = target_slice_idx.astype(jnp.int32).reshape(GRID, 1, TBLK)
    out = _dist_call(sums, cnts, target_embeddings, tidx_r)
    return out[0, 0]


# speedup vs baseline: 9.2674x; 1.3853x over previous
import jax
import jax.numpy as jnp
from jax import lax
from jax.experimental import pallas as pl
from jax.experimental.pallas import tpu as pltpu
from jax.experimental.pallas import tpu_sc as plsc

N_SRC = 100000
D = 128
NSEC = 100
NSEC_PAD = 128
B = 16384
SLICE_RANGE = 1000

NC = 2
NS = 16
NW = NC * NS

CHUNK = 400
SUB = 100
NSUB = CHUNK // SUB
NCHUNKS = N_SRC // CHUNK
NCH_PER_W = -(-NCHUNKS // NW)

HBLK = 12544
HGRID = -(-N_SRC // HBLK)
N_SRC_PAD = HGRID * HBLK

TBLK = 2048
GRID = B // TBLK


def _seg_sum_body(src_emb, sectors, zeros_acc, out_sums,
                  rows_v0, rows_v1, idx_v0, idx_v1, acc_sh,
                  lsem0, lsem1, isem0, isem1):
    c = lax.axis_index("c")
    s = lax.axis_index("s")
    wid = c * NS + s

    @pl.when(s == 0)
    def _():
        pltpu.sync_copy(zeros_acc, acc_sh)

    plsc.subcore_barrier()

    rows = (rows_v0, rows_v1)
    idxs = (idx_v0, idx_v1)
    lsems = (lsem0, lsem1)
    isems = (isem0, isem1)

    def start(k):
        ci = k * NW + wid
        b = k % 2
        pltpu.async_copy(src_emb.at[pl.ds(ci * CHUNK, CHUNK)], rows[b],
                         lsems[b])
        pltpu.async_copy(sectors.at[ci], idxs[b], isems[b])

    def consume(k):
        ci = k * NW + wid
        b = k % 2
        pltpu.make_async_copy(src_emb.at[pl.ds(ci * CHUNK, CHUNK)], rows[b],
                              lsems[b]).wait()
        pltpu.make_async_copy(sectors.at[ci], idxs[b], isems[b]).wait()
        for j in range(NSUB):
            pltpu.sync_copy(rows[b].at[pl.ds(j * SUB, SUB)],
                            acc_sh.at[idxs[b].at[j]], add=True)

    start(0)
    for k in range(NCH_PER_W):
        if k + 1 < NCH_PER_W:
            if k + 1 == NCH_PER_W - 1:
                @pl.when((k + 1) * NW + wid < NCHUNKS)
                def _(k=k):
                    start(k + 1)
            else:
                start(k + 1)
        if k == NCH_PER_W - 1:
            @pl.when(k * NW + wid < NCHUNKS)
            def _(k=k):
                consume(k)
        else:
            consume(k)

    plsc.subcore_barrier()

    @pl.when(s == 0)
    def _():
        pltpu.sync_copy(acc_sh, out_sums.at[c])


_seg_call_cache = []


def _seg_call(*args):
    if not _seg_call_cache:
        _seg_call_cache.append(pl.kernel(
            _seg_sum_body,
            out_type=jax.ShapeDtypeStruct((NC, NSEC_PAD, D), jnp.float32),
            mesh=plsc.VectorSubcoreMesh(core_axis_name="c",
                                        subcore_axis_name="s",
                                        num_cores=NC, num_subcores=NS),
            scratch_types=[
                pltpu.VMEM((CHUNK, D), jnp.float32),
                pltpu.VMEM((CHUNK, D), jnp.float32),
                pltpu.VMEM((NSUB, SUB), jnp.int32),
                pltpu.VMEM((NSUB, SUB), jnp.int32),
                pltpu.VMEM_SHARED((NSEC_PAD, D), jnp.float32),
                pltpu.SemaphoreType.DMA,
                pltpu.SemaphoreType.DMA,
                pltpu.SemaphoreType.DMA,
                pltpu.SemaphoreType.DMA,
            ],
        ))
    return _seg_call_cache[0](*args)


def _hist_body(sec_ref, out_ref):
    i = pl.program_id(0)

    @pl.when(i == 0)
    def _():
        out_ref[...] = jnp.zeros((1, NSEC_PAD), jnp.float32)

    sec = sec_ref[0, 0, :]
    onehot = (sec[:, None] == lax.broadcasted_iota(
        jnp.int32, (HBLK, NSEC_PAD), 1)).astype(jnp.float32)
    out_ref[...] += jnp.sum(onehot, axis=0).reshape(1, NSEC_PAD)


def _hist_call(sec_r):
    return pl.pallas_call(
        _hist_body,
        grid=(HGRID,),
        in_specs=[pl.BlockSpec((1, 1, HBLK), lambda i: (i, 0, 0))],
        out_specs=pl.BlockSpec((1, NSEC_PAD), lambda i: (0, 0)),
        out_shape=jax.ShapeDtypeStruct((1, NSEC_PAD), jnp.float32),
    )(sec_r)


def _dist_body(sums_ref, cnts_ref, temb_ref, tidx_ref, out_ref, centers_scr):
    i = pl.program_id(0)

    @pl.when(i == 0)
    def _():
        ssum = sums_ref[0] + sums_ref[1]
        cnt = cnts_ref[0]
        centers = ssum / jnp.maximum(cnt, 1.0)[:, None]
        centers_scr[...] = centers.astype(jnp.bfloat16)
        out_ref[...] = jnp.zeros((1, 1), jnp.float32)

    sec = tidx_ref[0, 0, :] // SLICE_RANGE
    onehot = (sec[:, None] == lax.broadcasted_iota(
        jnp.int32, (TBLK, NSEC_PAD), 1)).astype(jnp.bfloat16)
    cc = jnp.dot(onehot, centers_scr[...],
                 preferred_element_type=jnp.float32)
    diff = temb_ref[...] - cc + 1e-6
    dist = jnp.sqrt(jnp.sum(diff * diff, axis=-1))
    out_ref[...] += (jnp.sum(dist) * (1.0 / B)).reshape(1, 1)


def _dist_call(sums, cnts, temb, tidx_r):
    return pl.pallas_call(
        _dist_body,
        grid=(GRID,),
        in_specs=[
            pl.BlockSpec((NC, NSEC_PAD, D), lambda i: (0, 0, 0)),
            pl.BlockSpec((1, NSEC_PAD), lambda i: (0, 0)),
            pl.BlockSpec((TBLK, D), lambda i: (i, 0)),
            pl.BlockSpec((1, 1, TBLK), lambda i: (i, 0, 0)),
        ],
        out_specs=pl.BlockSpec((1, 1), lambda i: (0, 0)),
        out_shape=jax.ShapeDtypeStruct((1, 1), jnp.float32),
        scratch_shapes=[pltpu.VMEM((NSEC_PAD, D), jnp.bfloat16)],
    )(sums, cnts, temb, tidx_r)


def kernel(target_embeddings, target_slice_idx, source_embeddings,
           source_slice_idx, source_sectors):
    del source_slice_idx
    sec32 = source_sectors.astype(jnp.int32)
    sectors_r = sec32.reshape(NCHUNKS, NSUB, SUB)
    zeros_acc = jnp.zeros((NSEC_PAD, D), jnp.float32)

    sums = _seg_call(source_embeddings, sectors_r, zeros_acc)

    sec_pad = jnp.concatenate(
        [sec32, jnp.full((N_SRC_PAD - N_SRC,), NSEC_PAD - 1, jnp.int32)])
    cnts = _hist_call(sec_pad.reshape(HGRID, 1, HBLK))

    tidx_r = target_slice_idx.astype(jnp.int32).reshape(GRID, 1, TBLK)
    out = _dist_call(sums, cnts, target_embeddings, tidx_r)
    return out[0, 0]
